# baseline (speedup 1.0000x reference)
import functools

import jax
import jax.numpy as jnp
from jax.experimental import pallas as pl
from jax.experimental.pallas import tpu as pltpu

H = 128
AMPLIFY = 3.5
_F32 = jnp.float32


def _mm_kernel(x_ref, w_ref, b_ref, o_ref, *, relu_in, relu_out):
    x = x_ref[...]
    if relu_in:
        x = jnp.maximum(x, 0.0)
    y = jnp.dot(x, w_ref[...], preferred_element_type=_F32) + b_ref[...]
    if relu_out:
        y = jnp.maximum(y, 0.0)
    o_ref[...] = y


def _mm(x, w, b, *, relu_in=False, relu_out=False, block_rows=2000):
    n, k = x.shape
    m = w.shape[1]
    npad = -n % block_rows
    if npad:
        x = jnp.pad(x, ((0, npad), (0, 0)))
    nt = n + npad
    kern = functools.partial(_mm_kernel, relu_in=relu_in, relu_out=relu_out)
    out = pl.pallas_call(
        kern,
        grid=(nt // block_rows,),
        in_specs=[
            pl.BlockSpec((block_rows, k), lambda i: (i, 0)),
            pl.BlockSpec((k, m), lambda i: (0, 0)),
            pl.BlockSpec((1, m), lambda i: (0, 0)),
        ],
        out_specs=pl.BlockSpec((block_rows, m), lambda i: (i, 0)),
        out_shape=jax.ShapeDtypeStruct((nt, m), _F32),
    )(x, w, b.reshape(1, m))
    return out[:n] if npad else out


def _node_kernel(deg_ref, s_ref, ss_ref, mx_ref, mn_ref, u1w_ref, u1b_ref,
                 o_ref):
    deg = deg_ref[...]
    degc = jnp.maximum(deg, 1.0)
    mean = s_ref[...] / degc
    var = jnp.maximum(ss_ref[...] / degc - mean * mean, 0.0)
    std = jnp.sqrt(var + 1e-5)
    has = deg > 0.0
    zero = jnp.zeros_like(mean)
    mx = jnp.where(has, mx_ref[...], zero)
    mn = jnp.where(has, mn_ref[...], zero)
    agg = jnp.concatenate([mx, mn, mean, std], axis=1)
    logd = jnp.log(deg + 1.0)
    amp = logd * (1.0 / AMPLIFY)
    att = jnp.where(has, AMPLIFY / jnp.maximum(logd, 1e-5), zero)
    amp4 = jnp.concatenate([amp] * 4, axis=1)
    att4 = jnp.concatenate([att] * 4, axis=1)
    scaled = jnp.concatenate([agg, agg * amp4, agg * att4], axis=1)
    h = jnp.dot(scaled, u1w_ref[...], preferred_element_type=_F32) + u1b_ref[...]
    o_ref[...] = jnp.maximum(h, 0.0)


def _pad_rows(x, nt):
    return x if x.shape[0] == nt else jnp.pad(x, ((0, nt - x.shape[0]), (0, 0)))


def _node(degb, s, ss, mx, mn, u1w, u1b, *, block_rows=1000):
    n = s.shape[0]
    block_rows = min(block_rows, -(-n // 8) * 8)
    nt = -(-n // block_rows) * block_rows
    degb, s, ss, mx, mn = (_pad_rows(a, nt) for a in (degb, s, ss, mx, mn))
    spec = pl.BlockSpec((block_rows, H), lambda i: (i, 0))
    out = pl.pallas_call(
        _node_kernel,
        grid=(nt // block_rows,),
        in_specs=[spec, spec, spec, spec, spec,
                  pl.BlockSpec((12 * H, H), lambda i: (0, 0)),
                  pl.BlockSpec((1, H), lambda i: (0, 0))],
        out_specs=spec,
        out_shape=jax.ShapeDtypeStruct((nt, H), _F32),
    )(degb, s, ss, mx, mn, u1w, u1b.reshape(1, H))
    return out[:n] if nt != n else out


def _head_kernel(x_ref, w1_ref, b1_ref, w2_ref, b2_ref, w3_ref, b3_ref,
                 o_ref):
    h = jnp.dot(x_ref[...], w1_ref[...], preferred_element_type=_F32) + b1_ref[...]
    h = jnp.maximum(h, 0.0)
    h = jnp.dot(h, w2_ref[...], preferred_element_type=_F32) + b2_ref[...]
    h = jnp.maximum(h, 0.0)
    z = jnp.dot(h, w3_ref[...], preferred_element_type=_F32) + b3_ref[...]
    o_ref[...] = 1.0 / (1.0 + jnp.exp(-z))


def _head(x, w1, b1, w2, b2, w3, b3, *, block_rows=2000):
    n = x.shape[0]
    block_rows = min(block_rows, -(-n // 8) * 8)
    nt = -(-n // block_rows) * block_rows
    x = _pad_rows(x, nt)
    spec = pl.BlockSpec((block_rows, H), lambda i: (i, 0))
    wspec = pl.BlockSpec((H, H), lambda i: (0, 0))
    bspec = pl.BlockSpec((1, H), lambda i: (0, 0))
    out = pl.pallas_call(
        _head_kernel,
        grid=(nt // block_rows,),
        in_specs=[spec, wspec, bspec, wspec, bspec, wspec, bspec],
        out_specs=spec,
        out_shape=jax.ShapeDtypeStruct((nt, H), _F32),
    )(x, w1, b1.reshape(1, H), w2, b2.reshape(1, H), w3, b3.reshape(1, H))
    return out[:n] if nt != n else out


def _pna_hidden(At, Bn, tgt_s, nbr_s, n, degb, m2w, m2b, u1w, u1b):
    pre = jnp.take(At, tgt_s, axis=0) + jnp.take(Bn, nbr_s, axis=0)
    m = _mm(pre, m2w, m2b, relu_in=True, block_rows=2560)
    s = jax.ops.segment_sum(m, tgt_s, num_segments=n, indices_are_sorted=True)
    ss = jax.ops.segment_sum(m * m, tgt_s, num_segments=n,
                             indices_are_sorted=True)
    mx = jax.ops.segment_max(m, tgt_s, num_segments=n, indices_are_sorted=True)
    mn = -jax.ops.segment_max(-m, tgt_s, num_segments=n,
                              indices_are_sorted=True)
    return _node(degb, s, ss, mx, mn, u1w, u1b)


def kernel(demand, fac_init, adj, params):
    C = demand.shape[0]
    F = fac_init.shape[0]
    dst = adj[0]
    src = adj[1]

    perm_c = jnp.argsort(dst)
    dst_c = dst[perm_c]
    src_c = src[perm_c]
    perm_f = jnp.argsort(src)
    src_f = src[perm_f]
    dst_f = dst[perm_f]
    offs_c = jnp.searchsorted(dst_c, jnp.arange(C + 1, dtype=jnp.int32))
    deg_c = jnp.diff(offs_c).astype(_F32)
    offs_f = jnp.searchsorted(src_f, jnp.arange(F + 1, dtype=jnp.int32))
    deg_f = jnp.diff(offs_f).astype(_F32)
    degb_c = jnp.broadcast_to(deg_c[:, None], (C, H))
    degb_f = jnp.broadcast_to(deg_f[:, None], (F, H))

    p = params
    wce, bce = p["cus_emv"]["W"], p["cus_emv"]["b"]
    wfe, bfe = p["fac_emv"]["W"], p["fac_emv"]["b"]
    cp, fp = p["cus_pna"], p["fac_pna"]
    wt_c, wn_c, b1_c = cp["M1"]["W"][:H], cp["M1"]["W"][H:], cp["M1"]["b"]
    wt_f, wn_f, b1_f = fp["M1"]["W"][:H], fp["M1"]["W"][H:], fp["M1"]["b"]
    m2w_c, m2b_c = cp["M2"]["W"], cp["M2"]["b"]
    m2w_f, m2b_f = fp["M2"]["W"], fp["M2"]["b"]
    u1w_c, u1b_c = cp["U1"]["W"], cp["U1"]["b"]
    u1w_f, u1b_f = fp["U1"]["W"], fp["U1"]["b"]
    u2w_c, u2b_c = cp["U2"]["W"], cp["U2"]["b"]
    u2w_f, u2b_f = fp["U2"]["W"], fp["U2"]["b"]

    w_atc, b_atc = u2w_c @ wt_c, u2b_c @ wt_c + b1_c
    w_bnc, b_bnc = u2w_c @ wn_f, u2b_c @ wn_f
    w_atf, b_atf = u2w_f @ wt_f, u2b_f @ wt_f + b1_f
    w_bnf, b_bnf = u2w_f @ wn_c, u2b_f @ wn_c

    at_c = _mm(demand, wce @ wt_c, bce @ wt_c + b1_c)
    bn_c = _mm(demand, wce @ wn_f, bce @ wn_f)
    at_f = _mm(fac_init, wfe @ wt_f, bfe @ wt_f + b1_f)
    bn_f = _mm(fac_init, wfe @ wn_c, bfe @ wn_c)

    h_c = _pna_hidden(at_c, bn_f, dst_c, src_c, C, degb_c,
                      m2w_c, m2b_c, u1w_c, u1b_c)
    h_f = _pna_hidden(at_f, bn_c, src_f, dst_f, F, degb_f,
                      m2w_f, m2b_f, u1w_f, u1b_f)
    at_c = _mm(h_c, w_atc, b_atc)
    bn_c = _mm(h_c, w_bnc, b_bnc)
    at_f = _mm(h_f, w_atf, b_atf)
    bn_f = _mm(h_f, w_bnf, b_bnf)

    h_c = _pna_hidden(at_c, bn_f, dst_c, src_c, C, degb_c,
                      m2w_c, m2b_c, u1w_c, u1b_c)
    h_f = _pna_hidden(at_f, bn_c, src_f, dst_f, F, degb_f,
                      m2w_f, m2b_f, u1w_f, u1b_f)
    bn_c = _mm(h_c, w_bnc, b_bnc)
    at_f = _mm(h_f, w_atf, b_atf)

    h_f = _pna_hidden(at_f, bn_c, src_f, dst_f, F, degb_f,
                      m2w_f, m2b_f, u1w_f, u1b_f)
    f3 = _mm(h_f, u2w_f, u2b_f)

    f1w = jnp.zeros((H, H), _F32).at[:, :12].set(p["f1"]["W"])
    f1b = jnp.zeros((H,), _F32).at[:12].set(p["f1"]["b"])
    f2w = jnp.zeros((H, H), _F32).at[:12, :12].set(p["f2"]["W"])
    f2b = jnp.zeros((H,), _F32).at[:12].set(p["f2"]["b"])
    f3w = jnp.zeros((H, H), _F32).at[:12, :1].set(p["f3"]["W"])
    f3b = jnp.zeros((H,), _F32).at[:1].set(p["f3"]["b"])
    out = _head(f3, f1w, f1b, f2w, f2b, f3w, f3b)
    return out[:, :1]
